# Initial kernel scaffold; baseline (speedup 1.0000x reference)
import jax
import jax.numpy as jnp
from jax.experimental import pallas as pl


def kernel(x_pos, ix_in0, ix_out0, ix_in1, ix_out1, ix_in2, ix_out2, ix_in3, ix_out3):
    raise NotImplementedError("write your pallas kernel here")


# speedup vs baseline: 28.6412x; 28.6412x over previous
import functools

import jax
import jax.numpy as jnp
from jax import lax
from jax.experimental import pallas as pl
from jax.experimental.pallas import tpu as pltpu
from jax.experimental.pallas import tpu_sc as plsc

NB_VARS = 100000
N1, F0 = 400000, 4
N2, F1 = 100000, 8
N3, F2 = 20000, 5
N4, F3 = 5000, 4

NUM_WORKERS = 32
LANES = 16


def _mesh():
    return plsc.VectorSubcoreMesh(core_axis_name="c", subcore_axis_name="s")


def _wid():
    return lax.axis_index("s") * 2 + lax.axis_index("c")


def _encode_table(x_pos):
    ch = 2000
    nchunks = NB_VARS // ch

    @functools.partial(
        pl.kernel,
        out_type=jax.ShapeDtypeStruct((2 * NB_VARS,), jnp.float32),
        mesh=_mesh(),
        scratch_types=[
            pltpu.VMEM((ch,), jnp.float32),
            pltpu.VMEM((ch,), jnp.float32),
        ],
    )
    def k(pos_hbm, tab_hbm, pos_v, neg_v):
        w = _wid()

        @pl.loop(w, nchunks, step=NUM_WORKERS)
        def _(c):
            base = c * ch
            pltpu.sync_copy(pos_hbm.at[pl.ds(base, ch)], pos_v)

            @pl.loop(0, ch, step=LANES)
            def _(i):
                sl = pl.ds(i, LANES)
                neg_v[sl] = 1.0 - pos_v[sl]

            pltpu.sync_copy(pos_v, tab_hbm.at[pl.ds(base, ch)])
            pltpu.sync_copy(neg_v, tab_hbm.at[pl.ds(NB_VARS + base, ch)])

    return k(x_pos)


def _layer(table, idx_streams, n_pad, fan, is_prod, ch):
    nchunks = n_pad // ch

    @functools.partial(
        pl.kernel,
        out_type=jax.ShapeDtypeStruct((n_pad,), jnp.float32),
        mesh=_mesh(),
        scratch_types=(
            [pltpu.VMEM((ch,), jnp.int32) for _ in range(fan)]
            + [pltpu.VMEM((ch,), jnp.float32) for _ in range(fan)]
            + [
                pltpu.VMEM((ch,), jnp.float32),
                pltpu.SemaphoreType.DMA,
                pltpu.SemaphoreType.DMA,
            ]
        ),
    )
    def k(tab_hbm, *rest):
        idx_hbm = rest[:fan]
        out_hbm = rest[fan]
        rest = rest[fan + 1:]
        idx_v = rest[:fan]
        g_v = rest[fan:2 * fan]
        out_v, isem, gsem = rest[2 * fan:]
        w = _wid()

        @pl.loop(w, nchunks, step=NUM_WORKERS)
        def _(c):
            base = c * ch
            icopies = [
                pltpu.async_copy(idx_hbm[j].at[pl.ds(base, ch)], idx_v[j], isem)
                for j in range(fan)
            ]
            for cp in icopies:
                cp.wait()
            gcopies = [
                pltpu.async_copy(tab_hbm.at[idx_v[j]], g_v[j], gsem)
                for j in range(fan)
            ]
            for cp in gcopies:
                cp.wait()

            @pl.loop(0, ch, step=LANES)
            def _(i):
                sl = pl.ds(i, LANES)
                acc = g_v[0][sl]
                for j in range(1, fan):
                    acc = acc * g_v[j][sl] if is_prod else acc + g_v[j][sl]
                out_v[sl] = acc

            pltpu.sync_copy(out_v, out_hbm.at[pl.ds(base, ch)])

    return k(table, *idx_streams)


def _prep_idx(ix, n, fan, n_pad):
    idx_t = ix.reshape(n, fan).T
    if n_pad > n:
        idx_t = jnp.pad(idx_t, ((0, 0), (0, n_pad - n)))
    return [idx_t[j] for j in range(fan)]


def _pad_to(n, ch):
    return ((n + ch - 1) // ch) * ch


def kernel(x_pos, ix_in0, ix_out0, ix_in1, ix_out1, ix_in2, ix_out2,
           ix_in3, ix_out3):
    del ix_out0, ix_out1, ix_out2, ix_out3

    t = ix_in0 - 2
    idx0 = (t >> 1) + (t & 1) * NB_VARS

    ch0, ch1, ch2, ch3 = 2048, 512, 128, 128
    p1, p2, p3, p4 = (_pad_to(N1, ch0), _pad_to(N2, ch1),
                      _pad_to(N3, ch2), _pad_to(N4, ch3))
    idx_t0 = _prep_idx(idx0, N1, F0, p1)
    idx_t1 = _prep_idx(ix_in1, N2, F1, p2)
    idx_t2 = _prep_idx(ix_in2, N3, F2, p3)
    idx_t3 = _prep_idx(ix_in3, N4, F3, p4)

    tab = _encode_table(x_pos)
    l0 = _layer(tab, idx_t0, p1, F0, True, ch0)
    l1 = _layer(l0, idx_t1, p2, F1, False, ch1)
    l2 = _layer(l1, idx_t2, p3, F2, True, ch2)
    l3 = _layer(l2, idx_t3, p4, F3, False, ch3)
    return l3[:N4]


# speedup vs baseline: 217.3074x; 7.5872x over previous
import dataclasses
import functools

import jax
import jax.numpy as jnp
from jax import lax
from jax.experimental import pallas as pl
from jax.experimental.pallas import tpu as pltpu
from jax.experimental.pallas import tpu_sc as plsc

N1, F0 = 400000, 4
N2, F1 = 100000, 8
N3, F2 = 20000, 5
N4, F3 = 5000, 4
N4P = 5120

NUM_WORKERS = 32
LANES = 16


def _mesh():
    return plsc.VectorSubcoreMesh(core_axis_name="c", subcore_axis_name="s")


def _compiler_params():
    cp = pltpu.CompilerParams()
    if "needs_layout_passes" in pltpu.CompilerParams.__dataclass_fields__:
        cp = dataclasses.replace(cp, needs_layout_passes=False)
    return cp


def _wid():
    return lax.axis_index("s") * 2 + lax.axis_index("c")


def _reduce_chunk(idx_v, g_ref, out_v, ch, fan, is_prod, decode_literals):
    iota_f = lax.iota(jnp.int32, LANES) * fan

    @pl.loop(0, ch, step=LANES)
    def _(i):
        acc = None
        for j in range(fan):
            pos = iota_f + (i * fan + j)
            raw = plsc.load_gather(idx_v, [pos])
            if decode_literals:
                var = (raw >> 1) - 1
                g = plsc.load_gather(g_ref, [var])
                sf = (raw & 1).astype(jnp.float32)
                g = sf + (1.0 - 2.0 * sf) * g
            else:
                g = plsc.load_gather(g_ref, [raw])
            if acc is None:
                acc = g
            else:
                acc = acc * g if is_prod else acc + g
        out_v[pl.ds(i, LANES)] = acc


def _vmem_layer(table, idx, n_out, fan, is_prod, ch, decode_literals=False):
    nchunks = n_out // ch
    v = table.shape[0]

    @functools.partial(
        pl.kernel,
        out_type=jax.ShapeDtypeStruct((n_out,), jnp.float32),
        mesh=_mesh(),
        compiler_params=_compiler_params(),
        scratch_types=[
            pltpu.VMEM((v,), jnp.float32),
            pltpu.VMEM((ch * fan,), jnp.int32),
            pltpu.VMEM((ch,), jnp.float32),
        ],
    )
    def k(tab_hbm, idx_hbm, out_hbm, tab_v, idx_v, out_v):
        pltpu.sync_copy(tab_hbm, tab_v)
        w = _wid()

        @pl.loop(w, nchunks, step=NUM_WORKERS)
        def _(c):
            base = c * ch
            pltpu.sync_copy(idx_hbm.at[pl.ds(base * fan, ch * fan)], idx_v)
            _reduce_chunk(idx_v, tab_v, out_v, ch, fan, is_prod,
                          decode_literals)
            pltpu.sync_copy(out_v, out_hbm.at[pl.ds(base, ch)])

    return k(table, idx)


def _stream_layer(table, idx, n_out, fan, is_prod, ch):
    nchunks = n_out // ch

    @functools.partial(
        pl.kernel,
        out_type=jax.ShapeDtypeStruct((n_out,), jnp.float32),
        mesh=_mesh(),
        compiler_params=_compiler_params(),
        scratch_types=[
            pltpu.VMEM((ch * fan,), jnp.int32),
            pltpu.VMEM((ch * fan,), jnp.float32),
            pltpu.VMEM((ch,), jnp.float32),
            pltpu.SemaphoreType.DMA,
        ],
    )
    def k(tab_hbm, idx_hbm, out_hbm, idx_v, g_v, out_v, sem):
        w = _wid()
        iota = lax.iota(jnp.int32, LANES)

        @pl.loop(w, nchunks, step=NUM_WORKERS)
        def _(c):
            base = c * ch
            pltpu.sync_copy(idx_hbm.at[pl.ds(base * fan, ch * fan)], idx_v)
            pltpu.async_copy(tab_hbm.at[idx_v], g_v, sem).wait()
            _reduce_chunk_positional(g_v, out_v, ch, fan, is_prod, iota)
            pltpu.sync_copy(out_v, out_hbm.at[pl.ds(base, ch)])

    return k(table, idx)


def _reduce_chunk_positional(g_v, out_v, ch, fan, is_prod, iota):
    iota_f = iota * fan

    @pl.loop(0, ch, step=LANES)
    def _(i):
        acc = None
        for j in range(fan):
            pos = iota_f + (i * fan + j)
            g = plsc.load_gather(g_v, [pos])
            if acc is None:
                acc = g
            else:
                acc = acc * g if is_prod else acc + g
        out_v[pl.ds(i, LANES)] = acc


def kernel(x_pos, ix_in0, ix_out0, ix_in1, ix_out1, ix_in2, ix_out2,
           ix_in3, ix_out3):
    del ix_out0, ix_out1, ix_out2, ix_out3

    ix3 = jnp.pad(ix_in3, (0, (N4P - N4) * F3))
    l0 = _vmem_layer(x_pos, ix_in0, N1, F0, True, 2000, decode_literals=True)
    l1 = _stream_layer(l0, ix_in1, N2, F1, False, 400)
    l2 = _vmem_layer(l1, ix_in2, N3, F2, True, 80)
    l3 = _vmem_layer(l2, ix3, N4P, F3, False, 160)
    return l3[:N4]


# speedup vs baseline: 282.5321x; 1.3001x over previous
import dataclasses
import functools

import jax
import jax.numpy as jnp
from jax import lax
from jax.experimental import pallas as pl
from jax.experimental.pallas import tpu as pltpu
from jax.experimental.pallas import tpu_sc as plsc

N1, F0 = 400000, 4
N2, F1 = 100000, 8
N3, F2 = 20000, 5
N4, F3 = 5000, 4
N1P = 409600
N3P = 20480
N4P = 5120

NUM_WORKERS = 32
LANES = 16


def _mesh():
    return plsc.VectorSubcoreMesh(core_axis_name="c", subcore_axis_name="s")


def _compiler_params():
    cp = pltpu.CompilerParams()
    if "needs_layout_passes" in pltpu.CompilerParams.__dataclass_fields__:
        cp = dataclasses.replace(cp, needs_layout_passes=False)
    return cp


def _wid():
    return lax.axis_index("s") * 2 + lax.axis_index("c")


def _reduce_chunk(idx_v, g_ref, out_v, ch, fan, is_prod, decode_literals,
                  idx_off=0):
    iota_f = lax.iota(jnp.int32, LANES) * fan

    @pl.loop(0, ch, step=LANES)
    def _(i):
        acc = None
        for j in range(fan):
            pos = iota_f + (idx_off + i * fan + j)
            raw = plsc.load_gather(idx_v, [pos])
            if decode_literals:
                var = (raw >> 1) - 1
                g = plsc.load_gather(g_ref, [var])
                sf = (raw & 1).astype(jnp.float32)
                g = sf + (1.0 - 2.0 * sf) * g
            else:
                g = plsc.load_gather(g_ref, [raw])
            if acc is None:
                acc = g
            else:
                acc = acc * g if is_prod else acc + g
        out_v[pl.ds(i, LANES)] = acc


def _reduce_positional(g_v, out_v, ch, fan, is_prod):
    iota_f = lax.iota(jnp.int32, LANES) * fan

    @pl.loop(0, ch, step=LANES)
    def _(i):
        acc = None
        for j in range(fan):
            pos = iota_f + (i * fan + j)
            g = plsc.load_gather(g_v, [pos])
            if acc is None:
                acc = g
            else:
                acc = acc * g if is_prod else acc + g
        out_v[pl.ds(i, LANES)] = acc


def _layer0(x_pos, idx):
    per_w = N1P // NUM_WORKERS
    nsub = 4
    ch = per_w // nsub
    fan = F0

    @functools.partial(
        pl.kernel,
        out_type=jax.ShapeDtypeStruct((N1P,), jnp.float32),
        mesh=_mesh(),
        compiler_params=_compiler_params(),
        scratch_types=[
            pltpu.VMEM((N2,), jnp.float32),
            pltpu.VMEM((ch * fan,), jnp.int32),
            pltpu.VMEM((ch * fan,), jnp.int32),
            pltpu.VMEM((ch,), jnp.float32),
            pltpu.SemaphoreType.DMA,
            pltpu.SemaphoreType.DMA,
        ],
    )
    def k(tab_hbm, idx_hbm, out_hbm, tab_v, idx_a, idx_b, out_v, tsem, isem):
        w = _wid()
        base = w * per_w
        tab_cp = pltpu.async_copy(tab_hbm, tab_v, tsem)
        bufs = [idx_a, idx_b]
        cps = [None] * nsub
        cps[0] = pltpu.async_copy(
            idx_hbm.at[pl.ds(base * fan, ch * fan)], bufs[0], isem)
        for t in range(nsub):
            if t + 1 < nsub:
                cps[t + 1] = pltpu.async_copy(
                    idx_hbm.at[pl.ds((base + (t + 1) * ch) * fan, ch * fan)],
                    bufs[(t + 1) % 2], isem)
            cps[t].wait()
            if t == 0:
                tab_cp.wait()
            _reduce_chunk(bufs[t % 2], tab_v, out_v, ch, fan, True, True)
            pltpu.sync_copy(out_v, out_hbm.at[pl.ds(base + t * ch, ch)])

    return k(x_pos, idx)


def _layer1(table, idx):
    fan = F1
    ch = 400
    nchunks = N2 // ch
    slice_w = N1P // LANES

    @functools.partial(
        pl.kernel,
        out_type=jax.ShapeDtypeStruct((N2,), jnp.float32),
        mesh=_mesh(),
        compiler_params=_compiler_params(),
        scratch_types=[
            pltpu.VMEM_SHARED((N1P,), jnp.float32),
            pltpu.VMEM((ch * fan,), jnp.int32),
            pltpu.VMEM((ch * fan,), jnp.float32),
            pltpu.VMEM((ch,), jnp.float32),
            pltpu.SemaphoreType.DMA,
        ],
    )
    def k(tab_hbm, idx_hbm, out_hbm, tab_s, idx_v, g_v, out_v, sem):
        sid = lax.axis_index("s")
        pltpu.sync_copy(tab_hbm.at[pl.ds(sid * slice_w, slice_w)],
                        tab_s.at[pl.ds(sid * slice_w, slice_w)])
        plsc.subcore_barrier()
        w = _wid()

        @pl.loop(w, nchunks, step=NUM_WORKERS)
        def _(c):
            base = c * ch
            pltpu.sync_copy(idx_hbm.at[pl.ds(base * fan, ch * fan)], idx_v)
            pltpu.async_copy(tab_s.at[idx_v], g_v, sem).wait()
            _reduce_positional(g_v, out_v, ch, fan, False)
            pltpu.sync_copy(out_v, out_hbm.at[pl.ds(base, ch)])

    return k(table, idx)


def _stream_layer(table, idx, n_out, fan, is_prod, ch):
    nchunks = n_out // ch

    @functools.partial(
        pl.kernel,
        out_type=jax.ShapeDtypeStruct((n_out,), jnp.float32),
        mesh=_mesh(),
        compiler_params=_compiler_params(),
        scratch_types=[
            pltpu.VMEM((ch * fan,), jnp.int32),
            pltpu.VMEM((ch * fan,), jnp.float32),
            pltpu.VMEM((ch,), jnp.float32),
            pltpu.SemaphoreType.DMA,
        ],
    )
    def k(tab_hbm, idx_hbm, out_hbm, idx_v, g_v, out_v, sem):
        w = _wid()

        @pl.loop(w, nchunks, step=NUM_WORKERS)
        def _(c):
            base = c * ch
            pltpu.sync_copy(idx_hbm.at[pl.ds(base * fan, ch * fan)], idx_v)
            pltpu.async_copy(tab_hbm.at[idx_v], g_v, sem).wait()
            _reduce_positional(g_v, out_v, ch, fan, is_prod)
            pltpu.sync_copy(out_v, out_hbm.at[pl.ds(base, ch)])

    return k(table, idx)


def _vmem_layer(table, idx, n_out, fan, is_prod, ch):
    nchunks = n_out // ch
    v = table.shape[0]

    @functools.partial(
        pl.kernel,
        out_type=jax.ShapeDtypeStruct((n_out,), jnp.float32),
        mesh=_mesh(),
        compiler_params=_compiler_params(),
        scratch_types=[
            pltpu.VMEM((v,), jnp.float32),
            pltpu.VMEM((ch * fan,), jnp.int32),
            pltpu.VMEM((ch,), jnp.float32),
        ],
    )
    def k(tab_hbm, idx_hbm, out_hbm, tab_v, idx_v, out_v):
        pltpu.sync_copy(tab_hbm, tab_v)
        w = _wid()

        @pl.loop(w, nchunks, step=NUM_WORKERS)
        def _(c):
            base = c * ch
            pltpu.sync_copy(idx_hbm.at[pl.ds(base * fan, ch * fan)], idx_v)
            _reduce_chunk(idx_v, tab_v, out_v, ch, fan, is_prod, False)
            pltpu.sync_copy(out_v, out_hbm.at[pl.ds(base, ch)])

    return k(table, idx)


def kernel(x_pos, ix_in0, ix_out0, ix_in1, ix_out1, ix_in2, ix_out2,
           ix_in3, ix_out3):
    del ix_out0, ix_out1, ix_out2, ix_out3

    ix0 = jnp.pad(ix_in0, (0, (N1P - N1) * F0), constant_values=2)
    ix2 = jnp.pad(ix_in2, (0, (N3P - N3) * F2))
    ix3 = jnp.pad(ix_in3, (0, (N4P - N4) * F3))

    l0 = _layer0(x_pos, ix0)
    l1 = _layer1(l0, ix_in1)
    l2 = _stream_layer(l1, ix2, N3P, F2, True, 640)
    l3 = _vmem_layer(l2, ix3, N4P, F3, False, 160)
    return l3[:N4]


# speedup vs baseline: 306.0946x; 1.0834x over previous
import dataclasses
import functools

import jax
import jax.numpy as jnp
from jax import lax
from jax.experimental import pallas as pl
from jax.experimental.pallas import tpu as pltpu
from jax.experimental.pallas import tpu_sc as plsc

N1, F0 = 400000, 4
N2, F1 = 100000, 8
N3, F2 = 20000, 5
N4, F3 = 5000, 4
N1P = 409600
N2P = 102400
N3P = 20480
N4P = 5120

NUM_WORKERS = 32
LANES = 16
UNROLL = 2


def _mesh():
    return plsc.VectorSubcoreMesh(core_axis_name="c", subcore_axis_name="s")


def _compiler_params():
    cp = pltpu.CompilerParams()
    if "needs_layout_passes" in pltpu.CompilerParams.__dataclass_fields__:
        cp = dataclasses.replace(cp, needs_layout_passes=False)
    return cp


def _wid():
    return lax.axis_index("s") * 2 + lax.axis_index("c")


def _reduce_chunk(idx_v, g_ref, out_v, ch, fan, is_prod, decode_literals):
    iota_f = lax.iota(jnp.int32, LANES) * fan

    @pl.loop(0, ch, step=LANES * UNROLL)
    def _(i):
        for u in range(UNROLL):
            iu = i + u * LANES
            acc = None
            for j in range(fan):
                pos = iota_f + (iu * fan + j)
                raw = plsc.load_gather(idx_v, [pos])
                if decode_literals:
                    var = (raw >> 1) - 1
                    g = plsc.load_gather(g_ref, [var])
                    sf = (raw & 1).astype(jnp.float32)
                    g = sf + (1.0 - 2.0 * sf) * g
                else:
                    g = plsc.load_gather(g_ref, [raw])
                if acc is None:
                    acc = g
                else:
                    acc = acc * g if is_prod else acc + g
            out_v[pl.ds(iu, LANES)] = acc


def _reduce_positional(g_v, out_v, ch, fan, is_prod):
    iota_f = lax.iota(jnp.int32, LANES) * fan

    @pl.loop(0, ch, step=LANES * UNROLL)
    def _(i):
        for u in range(UNROLL):
            iu = i + u * LANES
            acc = None
            for j in range(fan):
                pos = iota_f + (iu * fan + j)
                g = plsc.load_gather(g_v, [pos])
                if acc is None:
                    acc = g
                else:
                    acc = acc * g if is_prod else acc + g
            out_v[pl.ds(iu, LANES)] = acc


def _layer0(x_pos, idx):
    per_w = N1P // NUM_WORKERS
    nsub = 4
    ch = per_w // nsub
    fan = F0

    @functools.partial(
        pl.kernel,
        out_type=jax.ShapeDtypeStruct((N1P,), jnp.float32),
        mesh=_mesh(),
        compiler_params=_compiler_params(),
        scratch_types=[
            pltpu.VMEM((N2,), jnp.float32),
            pltpu.VMEM((ch * fan,), jnp.int32),
            pltpu.VMEM((ch * fan,), jnp.int32),
            pltpu.VMEM((ch,), jnp.float32),
            pltpu.SemaphoreType.DMA,
            pltpu.SemaphoreType.DMA,
        ],
    )
    def k(tab_hbm, idx_hbm, out_hbm, tab_v, idx_a, idx_b, out_v, tsem, isem):
        w = _wid()
        base = w * per_w
        tab_cp = pltpu.async_copy(tab_hbm, tab_v, tsem)
        bufs = [idx_a, idx_b]
        cps = [None] * nsub
        cps[0] = pltpu.async_copy(
            idx_hbm.at[pl.ds(base * fan, ch * fan)], bufs[0], isem)
        for t in range(nsub):
            if t + 1 < nsub:
                cps[t + 1] = pltpu.async_copy(
                    idx_hbm.at[pl.ds((base + (t + 1) * ch) * fan, ch * fan)],
                    bufs[(t + 1) % 2], isem)
            cps[t].wait()
            if t == 0:
                tab_cp.wait()
            _reduce_chunk(bufs[t % 2], tab_v, out_v, ch, fan, True, True)
            pltpu.sync_copy(out_v, out_hbm.at[pl.ds(base + t * ch, ch)])

    return k(x_pos, idx)


def _layer1(table, idx):
    fan = F1
    per_w = N2P // NUM_WORKERS
    nsub = 4
    ch = per_w // nsub
    cw = ch * fan
    slice_w = N1P // LANES

    @functools.partial(
        pl.kernel,
        out_type=jax.ShapeDtypeStruct((N2P,), jnp.float32),
        mesh=_mesh(),
        compiler_params=_compiler_params(),
        scratch_types=[
            pltpu.VMEM_SHARED((N1P,), jnp.float32),
            pltpu.VMEM((cw,), jnp.int32),
            pltpu.VMEM((cw,), jnp.int32),
            pltpu.VMEM((cw,), jnp.float32),
            pltpu.VMEM((cw,), jnp.float32),
            pltpu.VMEM((ch,), jnp.float32),
            pltpu.SemaphoreType.DMA,
            pltpu.SemaphoreType.DMA,
        ],
    )
    def k(tab_hbm, idx_hbm, out_hbm, tab_s, ia, ib_, ga, gb_, out_v, isem,
          gsem):
        sid = lax.axis_index("s")
        pltpu.sync_copy(tab_hbm.at[pl.ds(sid * slice_w, slice_w)],
                        tab_s.at[pl.ds(sid * slice_w, slice_w)])
        plsc.subcore_barrier()
        w = _wid()
        base = w * per_w
        ib = [ia, ib_]
        gb = [ga, gb_]
        icp = [None] * nsub
        gcp = [None] * nsub
        icp[0] = pltpu.async_copy(
            idx_hbm.at[pl.ds(base * fan, cw)], ib[0], isem)
        icp[1] = pltpu.async_copy(
            idx_hbm.at[pl.ds(base * fan + cw, cw)], ib[1], isem)
        icp[0].wait()
        gcp[0] = pltpu.async_copy(tab_s.at[ib[0]], gb[0], gsem)
        for t in range(nsub):
            gcp[t].wait()
            if t + 2 < nsub:
                icp[t + 2] = pltpu.async_copy(
                    idx_hbm.at[pl.ds(base * fan + (t + 2) * cw, cw)],
                    ib[t % 2], isem)
            if t + 1 < nsub:
                icp[t + 1].wait()
                gcp[t + 1] = pltpu.async_copy(
                    tab_s.at[ib[(t + 1) % 2]], gb[(t + 1) % 2], gsem)
            _reduce_positional(gb[t % 2], out_v, ch, fan, False)
            pltpu.sync_copy(out_v, out_hbm.at[pl.ds(base + t * ch, ch)])

    return k(table, idx)


def _spmem_layer(table, idx, tab_len, n_out, fan, is_prod):
    ch = n_out // NUM_WORKERS
    slice_w = tab_len // LANES

    @functools.partial(
        pl.kernel,
        out_type=jax.ShapeDtypeStruct((n_out,), jnp.float32),
        mesh=_mesh(),
        compiler_params=_compiler_params(),
        scratch_types=[
            pltpu.VMEM_SHARED((tab_len,), jnp.float32),
            pltpu.VMEM((ch * fan,), jnp.int32),
            pltpu.VMEM((ch * fan,), jnp.float32),
            pltpu.VMEM((ch,), jnp.float32),
            pltpu.SemaphoreType.DMA,
        ],
    )
    def k(tab_hbm, idx_hbm, out_hbm, tab_s, idx_v, g_v, out_v, sem):
        sid = lax.axis_index("s")
        pltpu.sync_copy(tab_hbm.at[pl.ds(sid * slice_w, slice_w)],
                        tab_s.at[pl.ds(sid * slice_w, slice_w)])
        icp = pltpu.async_copy(
            idx_hbm.at[pl.ds(_wid() * ch * fan, ch * fan)], idx_v, sem)
        plsc.subcore_barrier()
        icp.wait()
        pltpu.async_copy(tab_s.at[idx_v], g_v, sem).wait()
        _reduce_positional(g_v, out_v, ch, fan, is_prod)
        pltpu.sync_copy(out_v, out_hbm.at[pl.ds(_wid() * ch, ch)])

    return k(table, idx)


def _vmem_layer(table, idx, n_out, fan, is_prod, ch):
    nchunks = n_out // ch
    v = table.shape[0]

    @functools.partial(
        pl.kernel,
        out_type=jax.ShapeDtypeStruct((n_out,), jnp.float32),
        mesh=_mesh(),
        compiler_params=_compiler_params(),
        scratch_types=[
            pltpu.VMEM((v,), jnp.float32),
            pltpu.VMEM((ch * fan,), jnp.int32),
            pltpu.VMEM((ch,), jnp.float32),
        ],
    )
    def k(tab_hbm, idx_hbm, out_hbm, tab_v, idx_v, out_v):
        pltpu.sync_copy(tab_hbm, tab_v)
        w = _wid()

        @pl.loop(w, nchunks, step=NUM_WORKERS)
        def _(c):
            base = c * ch
            pltpu.sync_copy(idx_hbm.at[pl.ds(base * fan, ch * fan)], idx_v)
            _reduce_chunk(idx_v, tab_v, out_v, ch, fan, is_prod, False)
            pltpu.sync_copy(out_v, out_hbm.at[pl.ds(base, ch)])

    return k(table, idx)


def kernel(x_pos, ix_in0, ix_out0, ix_in1, ix_out1, ix_in2, ix_out2,
           ix_in3, ix_out3):
    del ix_out0, ix_out1, ix_out2, ix_out3

    ix0 = jnp.pad(ix_in0, (0, (N1P - N1) * F0), constant_values=2)
    ix1 = jnp.pad(ix_in1, (0, (N2P - N2) * F1))
    ix2 = jnp.pad(ix_in2, (0, (N3P - N3) * F2))
    ix3 = jnp.pad(ix_in3, (0, (N4P - N4) * F3))

    l0 = _layer0(x_pos, ix0)
    l1 = _layer1(l0, ix1)
    l2 = _spmem_layer(l1, ix2, N2P, N3P, F2, True)
    l3 = _vmem_layer(l2, ix3, N4P, F3, False, 160)
    return l3[:N4]


# speedup vs baseline: 316.8531x; 1.0351x over previous
import dataclasses
import functools

import jax
import jax.numpy as jnp
from jax import lax
from jax.experimental import pallas as pl
from jax.experimental.pallas import tpu as pltpu
from jax.experimental.pallas import tpu_sc as plsc

N1, F0 = 400000, 4
N2, F1 = 100000, 8
N3, F2 = 20000, 5
N4, F3 = 5000, 4
N1P = 409600
N2P = 102400
N3P = 20480
N4P = 5120

NUM_WORKERS = 32
LANES = 16
UNROLL = 2


def _mesh():
    return plsc.VectorSubcoreMesh(core_axis_name="c", subcore_axis_name="s")


def _compiler_params():
    cp = pltpu.CompilerParams()
    if "needs_layout_passes" in pltpu.CompilerParams.__dataclass_fields__:
        cp = dataclasses.replace(cp, needs_layout_passes=False)
    return cp


def _wid():
    return lax.axis_index("s") * 2 + lax.axis_index("c")


def _reduce_chunk(idx_v, g_ref, out_v, ch, fan, is_prod, decode_literals):
    iota_f = lax.iota(jnp.int32, LANES) * fan

    @pl.loop(0, ch, step=LANES * UNROLL)
    def _(i):
        for u in range(UNROLL):
            iu = i + u * LANES
            acc = None
            for j in range(fan):
                pos = iota_f + (iu * fan + j)
                raw = plsc.load_gather(idx_v, [pos])
                if decode_literals:
                    var = (raw >> 1) - 1
                    g = plsc.load_gather(g_ref, [var])
                    sf = (raw & 1).astype(jnp.float32)
                    g = jnp.abs(g - sf)
                else:
                    g = plsc.load_gather(g_ref, [raw])
                if acc is None:
                    acc = g
                else:
                    acc = acc * g if is_prod else acc + g
            out_v[pl.ds(iu, LANES)] = acc


def _reduce_positional(g_v, out_v, ch, fan, is_prod):
    iota_f = lax.iota(jnp.int32, LANES) * fan

    @pl.loop(0, ch, step=LANES * UNROLL)
    def _(i):
        for u in range(UNROLL):
            iu = i + u * LANES
            acc = None
            for j in range(fan):
                pos = iota_f + (iu * fan + j)
                g = plsc.load_gather(g_v, [pos])
                if acc is None:
                    acc = g
                else:
                    acc = acc * g if is_prod else acc + g
            out_v[pl.ds(iu, LANES)] = acc


def _layer0(x_pos, idx):
    per_w = N1P // NUM_WORKERS
    ch = 1600
    fan = F0
    nsub_full = per_w // ch
    nsub_tail = (N1 - (NUM_WORKERS - 1) * per_w) // ch

    @functools.partial(
        pl.kernel,
        out_type=jax.ShapeDtypeStruct((N1P,), jnp.float32),
        mesh=_mesh(),
        compiler_params=_compiler_params(),
        scratch_types=[
            pltpu.VMEM((N2,), jnp.float32),
            pltpu.VMEM((ch * fan,), jnp.int32),
            pltpu.VMEM((ch * fan,), jnp.int32),
            pltpu.VMEM((ch,), jnp.float32),
            pltpu.VMEM((ch,), jnp.float32),
            pltpu.SemaphoreType.DMA,
            pltpu.SemaphoreType.DMA,
            pltpu.SemaphoreType.DMA,
            pltpu.SemaphoreType.DMA,
            pltpu.SemaphoreType.DMA,
        ],
    )
    def k(tab_hbm, idx_hbm, out_hbm, tab_v, ia, ib_, oa, ob_, tsem,
          isem_a, isem_b, osem_a, osem_b):
        w = _wid()
        base = w * per_w
        ib = [ia, ib_]
        ob = [oa, ob_]
        isems = [isem_a, isem_b]
        osems = [osem_a, osem_b]
        tab_cp = pltpu.async_copy(tab_hbm, tab_v, tsem)

        def span(nsub):
            icp = [None] * nsub
            ocp = [None] * nsub

            def fire_idx(t):
                return pltpu.async_copy(
                    idx_hbm.at[pl.ds((base + t * ch) * fan, ch * fan)],
                    ib[t % 2], isems[t % 2])

            icp[0] = fire_idx(0)
            if nsub > 1:
                icp[1] = fire_idx(1)
            for t in range(nsub):
                if t >= 1 and t + 1 < nsub:
                    icp[t + 1] = fire_idx(t + 1)
                icp[t].wait()
                if t == 0:
                    tab_cp.wait()
                if t >= 2:
                    ocp[t - 2].wait()
                _reduce_chunk(ib[t % 2], tab_v, ob[t % 2], ch, fan, True,
                              True)
                ocp[t] = pltpu.async_copy(
                    ob[t % 2], out_hbm.at[pl.ds(base + t * ch, ch)],
                    osems[t % 2])
            for t in range(max(nsub - 2, 0), nsub):
                ocp[t].wait()

        @pl.when(w < NUM_WORKERS - 1)
        def _():
            span(nsub_full)

        @pl.when(w == NUM_WORKERS - 1)
        def _():
            span(nsub_tail)

    return k(x_pos, idx)


def _layer1(table, idx):
    fan = F1
    per_w = N2P // NUM_WORKERS
    nsub_full = 4
    ch = per_w // nsub_full
    cw = ch * fan
    slice_w = N1P // LANES
    nsub_tail = (N2 - (NUM_WORKERS - 1) * per_w) // ch

    @functools.partial(
        pl.kernel,
        out_type=jax.ShapeDtypeStruct((N2P,), jnp.float32),
        mesh=_mesh(),
        compiler_params=_compiler_params(),
        scratch_types=[
            pltpu.VMEM_SHARED((N1P,), jnp.float32),
            pltpu.VMEM((cw,), jnp.int32),
            pltpu.VMEM((cw,), jnp.int32),
            pltpu.VMEM((cw,), jnp.float32),
            pltpu.VMEM((cw,), jnp.float32),
            pltpu.VMEM((ch,), jnp.float32),
            pltpu.VMEM((ch,), jnp.float32),
            pltpu.SemaphoreType.DMA,
            pltpu.SemaphoreType.DMA,
            pltpu.SemaphoreType.DMA,
            pltpu.SemaphoreType.DMA,
            pltpu.SemaphoreType.DMA,
            pltpu.SemaphoreType.DMA,
        ],
    )
    def k(tab_hbm, idx_hbm, out_hbm, tab_s, ia, ib_, ga, gb_, oa, ob_,
          isem_a, isem_b, gsem_a, gsem_b, osem_a, osem_b):
        sid = lax.axis_index("s")
        pltpu.sync_copy(tab_hbm.at[pl.ds(sid * slice_w, slice_w)],
                        tab_s.at[pl.ds(sid * slice_w, slice_w)])
        plsc.subcore_barrier()
        w = _wid()
        base = w * per_w
        ib = [ia, ib_]
        gb = [ga, gb_]
        ob = [oa, ob_]
        isems = [isem_a, isem_b]
        gsems = [gsem_a, gsem_b]
        osems = [osem_a, osem_b]

        def span(nsub):
            icp = [None] * nsub
            gcp = [None] * nsub
            ocp = [None] * nsub

            def fire_idx(t):
                return pltpu.async_copy(
                    idx_hbm.at[pl.ds(base * fan + t * cw, cw)],
                    ib[t % 2], isems[t % 2])

            def fire_gather(t):
                src = tab_s if t % 2 == 0 else tab_hbm
                return pltpu.async_copy(src.at[ib[t % 2]], gb[t % 2],
                                        gsems[t % 2])

            icp[0] = fire_idx(0)
            if nsub > 1:
                icp[1] = fire_idx(1)
            icp[0].wait()
            gcp[0] = fire_gather(0)
            for t in range(nsub):
                if t + 1 < nsub:
                    icp[t + 1].wait()
                    gcp[t + 1] = fire_gather(t + 1)
                gcp[t].wait()
                if t + 2 < nsub:
                    icp[t + 2] = fire_idx(t + 2)
                if t >= 2:
                    ocp[t - 2].wait()
                _reduce_positional(gb[t % 2], ob[t % 2], ch, fan, False)
                ocp[t] = pltpu.async_copy(
                    ob[t % 2], out_hbm.at[pl.ds(base + t * ch, ch)],
                    osems[t % 2])
            for t in range(max(nsub - 2, 0), nsub):
                ocp[t].wait()

        @pl.when(w < NUM_WORKERS - 1)
        def _():
            span(nsub_full)

        @pl.when(w == NUM_WORKERS - 1)
        def _():
            span(nsub_tail)

    return k(table, idx)


def _spmem_layer(table, idx, tab_len, n_out, fan, is_prod):
    ch = n_out // NUM_WORKERS
    slice_w = tab_len // LANES

    @functools.partial(
        pl.kernel,
        out_type=jax.ShapeDtypeStruct((n_out,), jnp.float32),
        mesh=_mesh(),
        compiler_params=_compiler_params(),
        scratch_types=[
            pltpu.VMEM_SHARED((tab_len,), jnp.float32),
            pltpu.VMEM((ch * fan,), jnp.int32),
            pltpu.VMEM((ch * fan,), jnp.float32),
            pltpu.VMEM((ch,), jnp.float32),
            pltpu.SemaphoreType.DMA,
        ],
    )
    def k(tab_hbm, idx_hbm, out_hbm, tab_s, idx_v, g_v, out_v, sem):
        sid = lax.axis_index("s")
        pltpu.sync_copy(tab_hbm.at[pl.ds(sid * slice_w, slice_w)],
                        tab_s.at[pl.ds(sid * slice_w, slice_w)])
        icp = pltpu.async_copy(
            idx_hbm.at[pl.ds(_wid() * ch * fan, ch * fan)], idx_v, sem)
        plsc.subcore_barrier()
        icp.wait()
        pltpu.async_copy(tab_s.at[idx_v], g_v, sem).wait()
        _reduce_positional(g_v, out_v, ch, fan, is_prod)
        pltpu.sync_copy(out_v, out_hbm.at[pl.ds(_wid() * ch, ch)])

    return k(table, idx)


def _vmem_layer(table, idx, n_out, fan, is_prod, ch):
    nchunks = n_out // ch
    v = table.shape[0]

    @functools.partial(
        pl.kernel,
        out_type=jax.ShapeDtypeStruct((n_out,), jnp.float32),
        mesh=_mesh(),
        compiler_params=_compiler_params(),
        scratch_types=[
            pltpu.VMEM((v,), jnp.float32),
            pltpu.VMEM((ch * fan,), jnp.int32),
            pltpu.VMEM((ch,), jnp.float32),
        ],
    )
    def k(tab_hbm, idx_hbm, out_hbm, tab_v, idx_v, out_v):
        pltpu.sync_copy(tab_hbm, tab_v)
        w = _wid()

        @pl.loop(w, nchunks, step=NUM_WORKERS)
        def _(c):
            base = c * ch
            pltpu.sync_copy(idx_hbm.at[pl.ds(base * fan, ch * fan)], idx_v)
            _reduce_chunk(idx_v, tab_v, out_v, ch, fan, is_prod, False)
            pltpu.sync_copy(out_v, out_hbm.at[pl.ds(base, ch)])

    return k(table, idx)


def kernel(x_pos, ix_in0, ix_out0, ix_in1, ix_out1, ix_in2, ix_out2,
           ix_in3, ix_out3):
    del ix_out0, ix_out1, ix_out2, ix_out3

    ix2 = jnp.pad(ix_in2, (0, (N3P - N3) * F2))
    ix3 = jnp.pad(ix_in3, (0, (N4P - N4) * F3))

    l0 = _layer0(x_pos, ix_in0)
    l1 = _layer1(l0, ix_in1)
    l2 = _spmem_layer(l1, ix2, N2P, N3P, F2, True)
    l3 = _vmem_layer(l2, ix3, N4P, F3, False, 160)
    return l3[:N4]


# speedup vs baseline: 334.2231x; 1.0548x over previous
import dataclasses
import functools

import jax
import jax.numpy as jnp
from jax import lax
from jax.experimental import pallas as pl
from jax.experimental.pallas import tpu as pltpu
from jax.experimental.pallas import tpu_sc as plsc

N1, F0 = 400000, 4
N2, F1 = 100000, 8
N3, F2 = 20000, 5
N4, F3 = 5000, 4
N1P = 409600
N2P = 102400
N3P = 20480
N4P = 5120

NUM_WORKERS = 32
LANES = 16
UNROLL = 2


def _mesh():
    return plsc.VectorSubcoreMesh(core_axis_name="c", subcore_axis_name="s")


def _compiler_params():
    cp = pltpu.CompilerParams()
    if "needs_layout_passes" in pltpu.CompilerParams.__dataclass_fields__:
        cp = dataclasses.replace(cp, needs_layout_passes=False)
    return cp


def _wid():
    return lax.axis_index("s") * 2 + lax.axis_index("c")


def _reduce_chunk(idx_v, g_ref, out_v, ch, fan, is_prod, decode_literals):
    iota_f = lax.iota(jnp.int32, LANES) * fan

    @pl.loop(0, ch, step=LANES * UNROLL)
    def _(i):
        for u in range(UNROLL):
            iu = i + u * LANES
            acc = None
            for j in range(fan):
                pos = iota_f + (iu * fan + j)
                raw = plsc.load_gather(idx_v, [pos])
                if decode_literals:
                    var = (raw >> 1) - 1
                    g = plsc.load_gather(g_ref, [var])
                    sf = (raw & 1).astype(jnp.float32)
                    g = jnp.abs(g - sf)
                else:
                    g = plsc.load_gather(g_ref, [raw])
                if acc is None:
                    acc = g
                else:
                    acc = acc * g if is_prod else acc + g
            out_v[pl.ds(iu, LANES)] = acc


def _reduce_positional(g_v, out_v, ch, fan, is_prod):
    iota_f = lax.iota(jnp.int32, LANES) * fan

    @pl.loop(0, ch, step=LANES * UNROLL)
    def _(i):
        for u in range(UNROLL):
            iu = i + u * LANES
            acc = None
            for j in range(fan):
                pos = iota_f + (iu * fan + j)
                g = plsc.load_gather(g_v, [pos])
                if acc is None:
                    acc = g
                else:
                    acc = acc * g if is_prod else acc + g
            out_v[pl.ds(iu, LANES)] = acc


def _layer0(x_pos, idx):
    per_w = N1P // NUM_WORKERS
    ch = 1600
    fan = F0
    nsub_full = per_w // ch
    nsub_tail = (N1 - (NUM_WORKERS - 1) * per_w) // ch

    @functools.partial(
        pl.kernel,
        out_type=jax.ShapeDtypeStruct((N1P,), jnp.float32),
        mesh=_mesh(),
        compiler_params=_compiler_params(),
        scratch_types=[
            pltpu.VMEM((N2,), jnp.float32),
            pltpu.VMEM((ch * fan,), jnp.int32),
            pltpu.VMEM((ch * fan,), jnp.int32),
            pltpu.VMEM((ch,), jnp.float32),
            pltpu.VMEM((ch,), jnp.float32),
            pltpu.SemaphoreType.DMA,
            pltpu.SemaphoreType.DMA,
            pltpu.SemaphoreType.DMA,
            pltpu.SemaphoreType.DMA,
            pltpu.SemaphoreType.DMA,
        ],
    )
    def k(tab_hbm, idx_hbm, out_hbm, tab_v, ia, ib_, oa, ob_, tsem,
          isem_a, isem_b, osem_a, osem_b):
        w = _wid()
        base = w * per_w
        ib = [ia, ib_]
        ob = [oa, ob_]
        isems = [isem_a, isem_b]
        osems = [osem_a, osem_b]
        tab_cp = pltpu.async_copy(tab_hbm, tab_v, tsem)

        def span(nsub):
            icp = [None] * nsub
            ocp = [None] * nsub

            def fire_idx(t):
                return pltpu.async_copy(
                    idx_hbm.at[pl.ds((base + t * ch) * fan, ch * fan)],
                    ib[t % 2], isems[t % 2])

            icp[0] = fire_idx(0)
            if nsub > 1:
                icp[1] = fire_idx(1)
            for t in range(nsub):
                if t >= 1 and t + 1 < nsub:
                    icp[t + 1] = fire_idx(t + 1)
                icp[t].wait()
                if t == 0:
                    tab_cp.wait()
                if t >= 2:
                    ocp[t - 2].wait()
                _reduce_chunk(ib[t % 2], tab_v, ob[t % 2], ch, fan, True,
                              True)
                ocp[t] = pltpu.async_copy(
                    ob[t % 2], out_hbm.at[pl.ds(base + t * ch, ch)],
                    osems[t % 2])
            for t in range(max(nsub - 2, 0), nsub):
                ocp[t].wait()

        @pl.when(w < NUM_WORKERS - 1)
        def _():
            span(nsub_full)

        @pl.when(w == NUM_WORKERS - 1)
        def _():
            span(nsub_tail)

    return k(x_pos, idx)


def _layer1(table, idx):
    fan = F1
    per_w = N2P // NUM_WORKERS
    nsub_full = 4
    ch = per_w // nsub_full
    cw = ch * fan
    slice_w = N1P // LANES
    nsub_tail = (N2 - (NUM_WORKERS - 1) * per_w) // ch

    @functools.partial(
        pl.kernel,
        out_type=jax.ShapeDtypeStruct((N2P,), jnp.float32),
        mesh=_mesh(),
        compiler_params=_compiler_params(),
        scratch_types=[
            pltpu.VMEM_SHARED((N1P,), jnp.float32),
            pltpu.VMEM((cw,), jnp.int32),
            pltpu.VMEM((cw,), jnp.int32),
            pltpu.VMEM((cw,), jnp.float32),
            pltpu.VMEM((cw,), jnp.float32),
            pltpu.VMEM((ch,), jnp.float32),
            pltpu.VMEM((ch,), jnp.float32),
            pltpu.SemaphoreType.DMA,
            pltpu.SemaphoreType.DMA,
            pltpu.SemaphoreType.DMA,
            pltpu.SemaphoreType.DMA,
            pltpu.SemaphoreType.DMA,
            pltpu.SemaphoreType.DMA,
        ],
    )
    def k(tab_hbm, idx_hbm, out_hbm, tab_s, ia, ib_, ga, gb_, oa, ob_,
          isem_a, isem_b, gsem_a, gsem_b, osem_a, osem_b):
        sid = lax.axis_index("s")
        pltpu.sync_copy(tab_hbm.at[pl.ds(sid * slice_w, slice_w)],
                        tab_s.at[pl.ds(sid * slice_w, slice_w)])
        plsc.subcore_barrier()
        w = _wid()
        base = w * per_w
        ib = [ia, ib_]
        gb = [ga, gb_]
        ob = [oa, ob_]
        isems = [isem_a, isem_b]
        gsems = [gsem_a, gsem_b]
        osems = [osem_a, osem_b]

        def span(nsub):
            icp = [None] * nsub
            gcp = [None] * nsub
            ocp = [None] * nsub

            def fire_idx(t):
                return pltpu.async_copy(
                    idx_hbm.at[pl.ds(base * fan + t * cw, cw)],
                    ib[t % 2], isems[t % 2])

            def fire_gather(t):
                src = tab_s if t % 2 == 0 else tab_hbm
                return pltpu.async_copy(src.at[ib[t % 2]], gb[t % 2],
                                        gsems[t % 2])

            icp[0] = fire_idx(0)
            if nsub > 1:
                icp[1] = fire_idx(1)
            icp[0].wait()
            gcp[0] = fire_gather(0)
            for t in range(nsub):
                if t + 1 < nsub:
                    icp[t + 1].wait()
                    gcp[t + 1] = fire_gather(t + 1)
                gcp[t].wait()
                if t + 2 < nsub:
                    icp[t + 2] = fire_idx(t + 2)
                if t >= 2:
                    ocp[t - 2].wait()
                _reduce_positional(gb[t % 2], ob[t % 2], ch, fan, False)
                ocp[t] = pltpu.async_copy(
                    ob[t % 2], out_hbm.at[pl.ds(base + t * ch, ch)],
                    osems[t % 2])
            for t in range(max(nsub - 2, 0), nsub):
                ocp[t].wait()

        @pl.when(w < NUM_WORKERS - 1)
        def _():
            span(nsub_full)

        @pl.when(w == NUM_WORKERS - 1)
        def _():
            span(nsub_tail)

    return k(table, idx)


def _layers23(table, idx2, idx3):
    ch2 = N3P // LANES
    ch3 = N4P // NUM_WORKERS
    slice_w = N2P // LANES

    @functools.partial(
        pl.kernel,
        out_type=jax.ShapeDtypeStruct((N4P,), jnp.float32),
        mesh=_mesh(),
        compiler_params=_compiler_params(),
        scratch_types=[
            pltpu.VMEM_SHARED((N2P,), jnp.float32),
            pltpu.VMEM_SHARED((N3P,), jnp.float32),
            pltpu.VMEM((ch2 * F2,), jnp.int32),
            pltpu.VMEM((ch2 * F2,), jnp.float32),
            pltpu.VMEM((ch2,), jnp.float32),
            pltpu.VMEM((ch3 * F3,), jnp.int32),
            pltpu.VMEM((ch3 * F3,), jnp.float32),
            pltpu.VMEM((ch3,), jnp.float32),
            pltpu.SemaphoreType.DMA,
            pltpu.SemaphoreType.DMA,
        ],
    )
    def k(tab_hbm, idx2_hbm, idx3_hbm, out_hbm, l1_s, l2_s, i2, g2, o2,
          i3, g3, o3, sem_a, sem_b):
        sid = lax.axis_index("s")
        core = lax.axis_index("c")
        base3 = core * (N4P // 2) + sid * ch3
        pltpu.sync_copy(tab_hbm.at[pl.ds(sid * slice_w, slice_w)],
                        l1_s.at[pl.ds(sid * slice_w, slice_w)])
        icp2 = pltpu.async_copy(
            idx2_hbm.at[pl.ds(sid * ch2 * F2, ch2 * F2)], i2, sem_a)
        icp3 = pltpu.async_copy(
            idx3_hbm.at[pl.ds(base3 * F3, ch3 * F3)], i3, sem_b)
        plsc.subcore_barrier()
        icp2.wait()
        pltpu.async_copy(l1_s.at[i2], g2, sem_a).wait()
        _reduce_positional(g2, o2, ch2, F2, True)
        pltpu.sync_copy(o2, l2_s.at[pl.ds(sid * ch2, ch2)])
        plsc.subcore_barrier()
        icp3.wait()
        pltpu.async_copy(l2_s.at[i3], g3, sem_b).wait()
        _reduce_positional(g3, o3, ch3, F3, False)
        pltpu.sync_copy(o3, out_hbm.at[pl.ds(base3, ch3)])

    return k(table, idx2, idx3)


def kernel(x_pos, ix_in0, ix_out0, ix_in1, ix_out1, ix_in2, ix_out2,
           ix_in3, ix_out3):
    del ix_out0, ix_out1, ix_out2, ix_out3

    ix2 = jnp.pad(ix_in2, (0, (N3P - N3) * F2))
    ix3 = jnp.pad(ix_in3, (0, (N4P - N4) * F3))

    l0 = _layer0(x_pos, ix_in0)
    l1 = _layer1(l0, ix_in1)
    l3 = _layers23(l1, ix2, ix3)
    return l3[:N4]
